# baseline (speedup 1.0000x reference)
import functools

import jax
import jax.numpy as jnp
from jax import lax
from jax.experimental import pallas as pl
from jax.experimental.pallas import tpu as pltpu
from jax.experimental.pallas import tpu_sc as plsc

_N = 10000
_E = 320000
_G = 64
_D = 128
_H = 128
_OUT = 64

_NC = 2
_NS = 16
_NW = _NC * _NS

_B = 1000
_GRID = _N // _B

_BN_SCALE = 1.0 / (1.0 + 1e-5) ** 0.5


def _make_sc_agg(n, d, e):
    epw = e // _NW
    chunk = 128
    nchunks = epw // chunk
    tailc = epw - nchunks * chunk
    nt = nchunks // 6
    rpt = 624
    tail = n - _NS * rpt
    mesh = plsc.VectorSubcoreMesh(core_axis_name="c", subcore_axis_name="s",
                                  num_cores=_NC, num_subcores=_NS)

    @functools.partial(
        pl.kernel,
        out_type=jax.ShapeDtypeStruct((_NC, n, d), jnp.float32),
        mesh=mesh,
        scratch_types=[
            pltpu.VMEM((chunk, d), jnp.float32),
            pltpu.VMEM((chunk, d), jnp.float32),
            pltpu.VMEM((chunk, d), jnp.float32),
            pltpu.VMEM((2, chunk), jnp.int32),
            pltpu.VMEM((2, chunk), jnp.int32),
            pltpu.VMEM((2, chunk), jnp.int32),
            pltpu.VMEM((2, chunk), jnp.int32),
            pltpu.VMEM((2, chunk), jnp.int32),
            pltpu.VMEM((2, chunk), jnp.int32),
            pltpu.VMEM((2, tailc), jnp.int32),
            pltpu.VMEM_SHARED((n, d), jnp.float32),
            pltpu.SemaphoreType.DMA,
            pltpu.SemaphoreType.DMA,
            pltpu.SemaphoreType.DMA,
            pltpu.SemaphoreType.DMA,
            pltpu.SemaphoreType.DMA,
            pltpu.SemaphoreType.DMA,
            pltpu.SemaphoreType.DMA,
            pltpu.SemaphoreType.DMA,
            pltpu.SemaphoreType.DMA,
            pltpu.SemaphoreType.DMA,
            pltpu.SemaphoreType.DMA,
            pltpu.SemaphoreType.DMA,
            pltpu.SemaphoreType.DMA,
            pltpu.SemaphoreType.DMA,
        ],
    )
    def agg(table_hbm, src_hbm, dst_hbm, out_hbm,
            rows0, rows1, rows2, idx0, idx1, idx2, idx3, idx4, idx5, tidx, acc,
            gsem0, gsem1, gsem2, ssem0, ssem1, ssem2,
            isem0, isem1, isem2, isem3, isem4, isem5, tsem, zsem):
        cid = lax.axis_index("c")
        sid = lax.axis_index("s")
        wid = cid * _NS + sid
        base = wid * epw

        bufs = (rows0, rows1, rows2)
        idxs = (idx0, idx1, idx2, idx3, idx4, idx5)
        gsems = (gsem0, gsem1, gsem2)
        ssems = (ssem0, ssem1, ssem2)
        isems = (isem0, isem1, isem2, isem3, isem4, isem5)

        def idx_fire(slot, c):
            off = base + c * chunk
            pltpu.async_copy(src_hbm.at[pl.ds(off, chunk)], idxs[slot].at[0],
                             isems[slot])
            pltpu.async_copy(dst_hbm.at[pl.ds(off, chunk)], idxs[slot].at[1],
                             isems[slot])

        def idx_wait(slot, c):
            off = base + c * chunk
            pltpu.make_async_copy(src_hbm.at[pl.ds(off, chunk)],
                                  idxs[slot].at[0], isems[slot]).wait()
            pltpu.make_async_copy(dst_hbm.at[pl.ds(off, chunk)],
                                  idxs[slot].at[1], isems[slot]).wait()

        for m in range(6):
            idx_fire(m, m)
        toff = base + nchunks * chunk
        pltpu.async_copy(src_hbm.at[pl.ds(toff, tailc)], tidx.at[0], tsem)
        pltpu.async_copy(dst_hbm.at[pl.ds(toff, tailc)], tidx.at[1], tsem)

        zr = 16
        def zrow(i, _):
            def zlane(l, _):
                rows0[i, pl.ds(l * 16, 16)] = jnp.zeros((16,), jnp.float32)
                return 0
            return lax.fori_loop(0, d // 16, zlane, 0)
        lax.fori_loop(0, zr, zrow, 0)
        zsrc = rows0.at[pl.ds(0, zr)]

        def zacc(i, _):
            pltpu.async_copy(zsrc, acc.at[pl.ds(sid * rpt + i * zr, zr)], zsem)
            return 0
        lax.fori_loop(0, rpt // zr, zacc, 0)

        @pl.when(sid == _NS - 1)
        def _():
            pltpu.async_copy(zsrc, acc.at[pl.ds(_NS * rpt, tail)], zsem)

        def zdrain(i, _):
            pltpu.make_async_copy(
                zsrc, acc.at[pl.ds(sid * rpt, zr)], zsem).wait()
            return 0
        lax.fori_loop(0, rpt // zr, zdrain, 0)

        @pl.when(sid == _NS - 1)
        def _():
            pltpu.make_async_copy(
                zsrc, acc.at[pl.ds(_NS * rpt, tail)], zsem).wait()

        plsc.subcore_barrier()

        def gather(slot, jslot):
            return pltpu.async_copy(
                table_hbm.at[idxs[slot].at[0]], bufs[jslot], gsems[jslot])

        def gwait(slot, jslot):
            pltpu.make_async_copy(
                table_hbm.at[idxs[slot].at[0]], bufs[jslot],
                gsems[jslot]).wait()

        def sfire(slot, jslot):
            pltpu.async_copy(bufs[jslot], acc.at[idxs[slot].at[1]],
                             ssems[jslot], add=True)

        def swait(slot, jslot):
            pltpu.make_async_copy(bufs[jslot], acc.at[idxs[slot].at[1]],
                                  ssems[jslot]).wait()

        for m in range(3):
            idx_wait(m, m)
            gather(m, m)

        def six(t, _):
            for k in range(6):
                c = 6 * t + k
                jslot = k % 3
                gwait(k, jslot)
                sfire(k, jslot)
                jp = (k + 2) % 3
                ip = (k + 5) % 6
                g = (k + 2) % 6
                if k == 0:
                    @pl.when(t > 0)
                    def _():
                        swait(ip, jp)
                        idx_wait(g, c + 2)
                        gather(g, jp)
                        idx_fire(ip, c + 5)
                elif k <= 3:
                    swait(ip, jp)
                    idx_wait(g, c + 2)
                    gather(g, jp)

                    @pl.when(t < nt - 1)
                    def _():
                        idx_fire(ip, c + 5)
                else:
                    swait(ip, jp)

                    @pl.when(t < nt - 1)
                    def _():
                        idx_wait(g, c + 2)
                        gather(g, jp)
                        idx_fire(ip, c + 5)
            return 0

        lax.fori_loop(0, nt, six, 0)
        swait(5, 2)

        pltpu.make_async_copy(src_hbm.at[pl.ds(toff, tailc)],
                              tidx.at[0], tsem).wait()
        pltpu.make_async_copy(dst_hbm.at[pl.ds(toff, tailc)],
                              tidx.at[1], tsem).wait()
        pltpu.async_copy(table_hbm.at[tidx.at[0]], rows0.at[pl.ds(0, tailc)],
                         gsem0)
        pltpu.make_async_copy(table_hbm.at[tidx.at[0]],
                              rows0.at[pl.ds(0, tailc)], gsem0).wait()
        pltpu.sync_copy(rows0.at[pl.ds(0, tailc)], acc.at[tidx.at[1]],
                        add=True)
        plsc.subcore_barrier()

        pltpu.sync_copy(
            acc.at[pl.ds(sid * rpt, rpt)],
            out_hbm.at[cid, pl.ds(sid * rpt, rpt)],
        )

        @pl.when(sid == _NS - 1)
        def _():
            pltpu.sync_copy(
                acc.at[pl.ds(_NS * rpt, tail)],
                out_hbm.at[cid, pl.ds(_NS * rpt, tail)],
            )

    return agg


_sc_agg_cache = []


def _sc_agg(table, src, dst):
    if not _sc_agg_cache:
        _sc_agg_cache.append(_make_sc_agg(_N, _D, _E))
    return _sc_agg_cache[0](table, src, dst)


def _gin_block(x_ref, a0_ref, a1_ref, w1_ref, b1_ref, g1_ref, t1_ref,
               w2_ref, b2_ref, g2_ref, t2_ref):
    z = x_ref[...] + a0_ref[0] + a1_ref[0]
    h = jnp.dot(z, w1_ref[...], preferred_element_type=jnp.float32) + b1_ref[...]
    h = h * (g1_ref[...] * _BN_SCALE) + t1_ref[...]
    h = jnp.maximum(h, 0.0)
    h = jnp.dot(h, w2_ref[...], preferred_element_type=jnp.float32) + b2_ref[...]
    h = h * (g2_ref[...] * _BN_SCALE) + t2_ref[...]
    return jnp.maximum(h, 0.0)


def _onehot(b3_ref):
    b = b3_ref[0, 0, :]
    gid = lax.broadcasted_iota(jnp.int32, (_G, _B), 0)
    return (b[None, :] == gid).astype(jnp.float32)


def _layer1_body(x_ref, a0_ref, a1_ref, b3_ref,
                 w1_ref, b1_ref, g1_ref, t1_ref,
                 w2_ref, b2_ref, g2_ref, t2_ref,
                 h_ref, sums_ref, cnt_ref):
    pid = pl.program_id(0)
    h = _gin_block(x_ref, a0_ref, a1_ref, w1_ref, b1_ref, g1_ref, t1_ref,
                   w2_ref, b2_ref, g2_ref, t2_ref)
    h_ref[...] = h
    oh = _onehot(b3_ref)

    @pl.when(pid == 0)
    def _():
        sums_ref[...] = jnp.zeros_like(sums_ref)
        cnt_ref[...] = jnp.zeros_like(cnt_ref)

    sums_ref[...] += jnp.dot(oh, h, preferred_element_type=jnp.float32)
    cnt_ref[...] += jnp.broadcast_to(
        jnp.sum(oh, axis=1, keepdims=True), cnt_ref.shape)


def _layer2_body(x_ref, a0_ref, a1_ref, b3_ref,
                 w1_ref, b1_ref, g1_ref, t1_ref,
                 w2_ref, b2_ref, g2_ref, t2_ref,
                 m1_ref, cnt_ref, wl0_ref, wl1_ref, bl_ref,
                 out_ref, m2_ref):
    pid = pl.program_id(0)
    h = _gin_block(x_ref, a0_ref, a1_ref, w1_ref, b1_ref, g1_ref, t1_ref,
                   w2_ref, b2_ref, g2_ref, t2_ref)
    oh = _onehot(b3_ref)

    @pl.when(pid == 0)
    def _():
        m2_ref[...] = jnp.zeros_like(m2_ref)

    m2_ref[...] += jnp.dot(oh, h, preferred_element_type=jnp.float32)

    @pl.when(pid == _GRID - 1)
    def _():
        c = jnp.maximum(cnt_ref[...], 1.0)
        m1 = m1_ref[...] / c
        m2 = m2_ref[...] / c
        out_ref[...] = (jnp.dot(m1, wl0_ref[0], preferred_element_type=jnp.float32)
                        + jnp.dot(m2, wl1_ref[0], preferred_element_type=jnp.float32)
                        + bl_ref[...])


_row_spec = pl.BlockSpec((_B, _D), lambda i: (i, 0))
_agg0_spec = pl.BlockSpec((1, _B, _D), lambda i: (0, i, 0))
_agg1_spec = pl.BlockSpec((1, _B, _D), lambda i: (1, i, 0))
_b3_spec = pl.BlockSpec((1, 1, _B), lambda i: (i, 0, 0))
_w_spec = pl.BlockSpec((_D, _H), lambda i: (0, 0))
_v_spec = pl.BlockSpec((1, _H), lambda i: (0, 0))
_g_spec = pl.BlockSpec((_G, _H), lambda i: (0, 0))

_layer1 = pl.pallas_call(
    _layer1_body,
    grid=(_GRID,),
    in_specs=[_row_spec, _agg0_spec, _agg1_spec, _b3_spec,
              _w_spec, _v_spec, _v_spec, _v_spec,
              _w_spec, _v_spec, _v_spec, _v_spec],
    out_specs=[_row_spec, _g_spec, _g_spec],
    out_shape=[jax.ShapeDtypeStruct((_N, _H), jnp.float32),
               jax.ShapeDtypeStruct((_G, _H), jnp.float32),
               jax.ShapeDtypeStruct((_G, _H), jnp.float32)],
)

_wl_spec = pl.BlockSpec((1, _H, _OUT), lambda i: (0, 0, 0))
_wl1_spec = pl.BlockSpec((1, _H, _OUT), lambda i: (1, 0, 0))
_bl_spec = pl.BlockSpec((1, _OUT), lambda i: (0, 0))

_layer2 = pl.pallas_call(
    _layer2_body,
    grid=(_GRID,),
    in_specs=[_row_spec, _agg0_spec, _agg1_spec, _b3_spec,
              _w_spec, _v_spec, _v_spec, _v_spec,
              _w_spec, _v_spec, _v_spec, _v_spec,
              _g_spec, _g_spec, _wl_spec, _wl1_spec, _bl_spec],
    out_specs=pl.BlockSpec((_G, _OUT), lambda i: (0, 0)),
    out_shape=jax.ShapeDtypeStruct((_G, _OUT), jnp.float32),
    scratch_shapes=[pltpu.VMEM((_G, _H), jnp.float32)],
)


def kernel(x, edge_index, batch,
           W1a, b1a, g1a, t1a, W2a, b2a, g2a, t2a,
           W1b, b1b, g1b, t1b, W2b, b2b, g2b, t2b,
           Wl, bl):
    src = edge_index[0]
    dst = edge_index[1]
    b3 = batch.reshape(_GRID, 1, _B)
    r = lambda v: v.reshape(1, -1)

    agg1 = _sc_agg(x, src, dst)
    h1, m1s, cnt = _layer1(x, agg1, agg1, b3,
                           W1a, r(b1a), r(g1a), r(t1a),
                           W2a, r(b2a), r(g2a), r(t2a))
    agg2 = _sc_agg(h1, src, dst)
    wlr = Wl.reshape(2, _H, _OUT)
    return _layer2(h1, agg2, agg2, b3,
                   W1b, r(b1b), r(g1b), r(t1b),
                   W2b, r(b2b), r(g2b), r(t2b),
                   m1s, cnt, wlr, wlr, r(bl))


# speedup vs baseline: 1.0305x; 1.0305x over previous
import functools

import jax
import jax.numpy as jnp
from jax import lax
from jax.experimental import pallas as pl
from jax.experimental.pallas import tpu as pltpu
from jax.experimental.pallas import tpu_sc as plsc

_N = 10000
_E = 320000
_G = 64
_D = 128
_H = 128
_OUT = 64

_NC = 2
_NS = 16
_NW = _NC * _NS

_B = 2000
_GRID = _N // _B

_BN_SCALE = 1.0 / (1.0 + 1e-5) ** 0.5


def _make_sc_agg(n, d, e):
    epw = e // _NW
    chunk = 128
    nchunks = epw // chunk
    tailc = epw - nchunks * chunk
    nt = nchunks // 6
    rpt = 624
    tail = n - _NS * rpt
    mesh = plsc.VectorSubcoreMesh(core_axis_name="c", subcore_axis_name="s",
                                  num_cores=_NC, num_subcores=_NS)

    @functools.partial(
        pl.kernel,
        out_type=jax.ShapeDtypeStruct((_NC, n, d), jnp.float32),
        mesh=mesh,
        scratch_types=[
            pltpu.VMEM((chunk, d), jnp.float32),
            pltpu.VMEM((chunk, d), jnp.float32),
            pltpu.VMEM((chunk, d), jnp.float32),
            pltpu.VMEM((2, chunk), jnp.int32),
            pltpu.VMEM((2, chunk), jnp.int32),
            pltpu.VMEM((2, chunk), jnp.int32),
            pltpu.VMEM((2, chunk), jnp.int32),
            pltpu.VMEM((2, chunk), jnp.int32),
            pltpu.VMEM((2, chunk), jnp.int32),
            pltpu.VMEM((2, tailc), jnp.int32),
            pltpu.VMEM_SHARED((n, d), jnp.float32),
            pltpu.SemaphoreType.DMA,
            pltpu.SemaphoreType.DMA,
            pltpu.SemaphoreType.DMA,
            pltpu.SemaphoreType.DMA,
            pltpu.SemaphoreType.DMA,
            pltpu.SemaphoreType.DMA,
            pltpu.SemaphoreType.DMA,
            pltpu.SemaphoreType.DMA,
            pltpu.SemaphoreType.DMA,
            pltpu.SemaphoreType.DMA,
            pltpu.SemaphoreType.DMA,
            pltpu.SemaphoreType.DMA,
            pltpu.SemaphoreType.DMA,
            pltpu.SemaphoreType.DMA,
        ],
    )
    def agg(table_hbm, src_hbm, dst_hbm, out_hbm,
            rows0, rows1, rows2, idx0, idx1, idx2, idx3, idx4, idx5, tidx, acc,
            gsem0, gsem1, gsem2, ssem0, ssem1, ssem2,
            isem0, isem1, isem2, isem3, isem4, isem5, tsem, zsem):
        cid = lax.axis_index("c")
        sid = lax.axis_index("s")
        wid = cid * _NS + sid
        base = wid * epw

        bufs = (rows0, rows1, rows2)
        idxs = (idx0, idx1, idx2, idx3, idx4, idx5)
        gsems = (gsem0, gsem1, gsem2)
        ssems = (ssem0, ssem1, ssem2)
        isems = (isem0, isem1, isem2, isem3, isem4, isem5)

        def idx_fire(slot, c):
            off = base + c * chunk
            pltpu.async_copy(src_hbm.at[pl.ds(off, chunk)], idxs[slot].at[0],
                             isems[slot])
            pltpu.async_copy(dst_hbm.at[pl.ds(off, chunk)], idxs[slot].at[1],
                             isems[slot])

        def idx_wait(slot, c):
            off = base + c * chunk
            pltpu.make_async_copy(src_hbm.at[pl.ds(off, chunk)],
                                  idxs[slot].at[0], isems[slot]).wait()
            pltpu.make_async_copy(dst_hbm.at[pl.ds(off, chunk)],
                                  idxs[slot].at[1], isems[slot]).wait()

        for m in range(6):
            idx_fire(m, m)
        toff = base + nchunks * chunk
        pltpu.async_copy(src_hbm.at[pl.ds(toff, tailc)], tidx.at[0], tsem)
        pltpu.async_copy(dst_hbm.at[pl.ds(toff, tailc)], tidx.at[1], tsem)

        zr = 16
        def zrow(i, _):
            def zlane(l, _):
                rows0[i, pl.ds(l * 16, 16)] = jnp.zeros((16,), jnp.float32)
                return 0
            return lax.fori_loop(0, d // 16, zlane, 0)
        lax.fori_loop(0, zr, zrow, 0)
        zsrc = rows0.at[pl.ds(0, zr)]

        def zacc(i, _):
            pltpu.async_copy(zsrc, acc.at[pl.ds(sid * rpt + i * zr, zr)], zsem)
            return 0
        lax.fori_loop(0, rpt // zr, zacc, 0)

        @pl.when(sid == _NS - 1)
        def _():
            pltpu.async_copy(zsrc, acc.at[pl.ds(_NS * rpt, tail)], zsem)

        def zdrain(i, _):
            pltpu.make_async_copy(
                zsrc, acc.at[pl.ds(sid * rpt, zr)], zsem).wait()
            return 0
        lax.fori_loop(0, rpt // zr, zdrain, 0)

        @pl.when(sid == _NS - 1)
        def _():
            pltpu.make_async_copy(
                zsrc, acc.at[pl.ds(_NS * rpt, tail)], zsem).wait()

        plsc.subcore_barrier()

        def gather(slot, jslot):
            return pltpu.async_copy(
                table_hbm.at[idxs[slot].at[0]], bufs[jslot], gsems[jslot])

        def gwait(slot, jslot):
            pltpu.make_async_copy(
                table_hbm.at[idxs[slot].at[0]], bufs[jslot],
                gsems[jslot]).wait()

        def sfire(slot, jslot):
            pltpu.async_copy(bufs[jslot], acc.at[idxs[slot].at[1]],
                             ssems[jslot], add=True)

        def swait(slot, jslot):
            pltpu.make_async_copy(bufs[jslot], acc.at[idxs[slot].at[1]],
                                  ssems[jslot]).wait()

        for m in range(3):
            idx_wait(m, m)
            gather(m, m)

        def six(t, _):
            for k in range(6):
                c = 6 * t + k
                jslot = k % 3
                gwait(k, jslot)
                sfire(k, jslot)
                jp = (k + 2) % 3
                ip = (k + 5) % 6
                g = (k + 2) % 6
                if k == 0:
                    @pl.when(t > 0)
                    def _():
                        swait(ip, jp)
                        idx_wait(g, c + 2)
                        gather(g, jp)
                        idx_fire(ip, c + 5)
                elif k <= 3:
                    swait(ip, jp)
                    idx_wait(g, c + 2)
                    gather(g, jp)

                    @pl.when(t < nt - 1)
                    def _():
                        idx_fire(ip, c + 5)
                else:
                    swait(ip, jp)

                    @pl.when(t < nt - 1)
                    def _():
                        idx_wait(g, c + 2)
                        gather(g, jp)
                        idx_fire(ip, c + 5)
            return 0

        lax.fori_loop(0, nt, six, 0)
        swait(5, 2)

        pltpu.make_async_copy(src_hbm.at[pl.ds(toff, tailc)],
                              tidx.at[0], tsem).wait()
        pltpu.make_async_copy(dst_hbm.at[pl.ds(toff, tailc)],
                              tidx.at[1], tsem).wait()
        pltpu.async_copy(table_hbm.at[tidx.at[0]], rows0.at[pl.ds(0, tailc)],
                         gsem0)
        pltpu.make_async_copy(table_hbm.at[tidx.at[0]],
                              rows0.at[pl.ds(0, tailc)], gsem0).wait()
        pltpu.sync_copy(rows0.at[pl.ds(0, tailc)], acc.at[tidx.at[1]],
                        add=True)
        plsc.subcore_barrier()

        pltpu.sync_copy(
            acc.at[pl.ds(sid * rpt, rpt)],
            out_hbm.at[cid, pl.ds(sid * rpt, rpt)],
        )

        @pl.when(sid == _NS - 1)
        def _():
            pltpu.sync_copy(
                acc.at[pl.ds(_NS * rpt, tail)],
                out_hbm.at[cid, pl.ds(_NS * rpt, tail)],
            )

    return agg


_sc_agg_cache = []


def _sc_agg(table, src, dst):
    if not _sc_agg_cache:
        _sc_agg_cache.append(_make_sc_agg(_N, _D, _E))
    return _sc_agg_cache[0](table, src, dst)


def _gin_block(x_ref, a0_ref, a1_ref, w1_ref, b1_ref, g1_ref, t1_ref,
               w2_ref, b2_ref, g2_ref, t2_ref):
    z = x_ref[...] + a0_ref[0] + a1_ref[0]
    h = jnp.dot(z, w1_ref[...], preferred_element_type=jnp.float32) + b1_ref[...]
    h = h * (g1_ref[...] * _BN_SCALE) + t1_ref[...]
    h = jnp.maximum(h, 0.0)
    h = jnp.dot(h, w2_ref[...], preferred_element_type=jnp.float32) + b2_ref[...]
    h = h * (g2_ref[...] * _BN_SCALE) + t2_ref[...]
    return jnp.maximum(h, 0.0)


def _onehot(b3_ref):
    b = b3_ref[0, 0, :]
    gid = lax.broadcasted_iota(jnp.int32, (_G, _B), 0)
    return (b[None, :] == gid).astype(jnp.float32)


def _layer1_body(x_ref, a0_ref, a1_ref, b3_ref,
                 w1_ref, b1_ref, g1_ref, t1_ref,
                 w2_ref, b2_ref, g2_ref, t2_ref,
                 h_ref, sums_ref, cnt_ref):
    pid = pl.program_id(0)
    h = _gin_block(x_ref, a0_ref, a1_ref, w1_ref, b1_ref, g1_ref, t1_ref,
                   w2_ref, b2_ref, g2_ref, t2_ref)
    h_ref[...] = h
    oh = _onehot(b3_ref)

    @pl.when(pid == 0)
    def _():
        sums_ref[...] = jnp.zeros_like(sums_ref)
        cnt_ref[...] = jnp.zeros_like(cnt_ref)

    sums_ref[...] += jnp.dot(oh, h, preferred_element_type=jnp.float32)
    cnt_ref[...] += jnp.broadcast_to(
        jnp.sum(oh, axis=1, keepdims=True), cnt_ref.shape)


def _layer2_body(x_ref, a0_ref, a1_ref, b3_ref,
                 w1_ref, b1_ref, g1_ref, t1_ref,
                 w2_ref, b2_ref, g2_ref, t2_ref,
                 m1_ref, cnt_ref, wl0_ref, wl1_ref, bl_ref,
                 out_ref, m2_ref):
    pid = pl.program_id(0)
    h = _gin_block(x_ref, a0_ref, a1_ref, w1_ref, b1_ref, g1_ref, t1_ref,
                   w2_ref, b2_ref, g2_ref, t2_ref)
    oh = _onehot(b3_ref)

    @pl.when(pid == 0)
    def _():
        m2_ref[...] = jnp.zeros_like(m2_ref)

    m2_ref[...] += jnp.dot(oh, h, preferred_element_type=jnp.float32)

    @pl.when(pid == _GRID - 1)
    def _():
        c = jnp.maximum(cnt_ref[...], 1.0)
        m1 = m1_ref[...] / c
        m2 = m2_ref[...] / c
        out_ref[...] = (jnp.dot(m1, wl0_ref[0], preferred_element_type=jnp.float32)
                        + jnp.dot(m2, wl1_ref[0], preferred_element_type=jnp.float32)
                        + bl_ref[...])


_row_spec = pl.BlockSpec((_B, _D), lambda i: (i, 0))
_agg0_spec = pl.BlockSpec((1, _B, _D), lambda i: (0, i, 0))
_agg1_spec = pl.BlockSpec((1, _B, _D), lambda i: (1, i, 0))
_b3_spec = pl.BlockSpec((1, 1, _B), lambda i: (i, 0, 0))
_w_spec = pl.BlockSpec((_D, _H), lambda i: (0, 0))
_v_spec = pl.BlockSpec((1, _H), lambda i: (0, 0))
_g_spec = pl.BlockSpec((_G, _H), lambda i: (0, 0))

_layer1 = pl.pallas_call(
    _layer1_body,
    grid=(_GRID,),
    in_specs=[_row_spec, _agg0_spec, _agg1_spec, _b3_spec,
              _w_spec, _v_spec, _v_spec, _v_spec,
              _w_spec, _v_spec, _v_spec, _v_spec],
    out_specs=[_row_spec, _g_spec, _g_spec],
    out_shape=[jax.ShapeDtypeStruct((_N, _H), jnp.float32),
               jax.ShapeDtypeStruct((_G, _H), jnp.float32),
               jax.ShapeDtypeStruct((_G, _H), jnp.float32)],
)

_wl_spec = pl.BlockSpec((1, _H, _OUT), lambda i: (0, 0, 0))
_wl1_spec = pl.BlockSpec((1, _H, _OUT), lambda i: (1, 0, 0))
_bl_spec = pl.BlockSpec((1, _OUT), lambda i: (0, 0))

_layer2 = pl.pallas_call(
    _layer2_body,
    grid=(_GRID,),
    in_specs=[_row_spec, _agg0_spec, _agg1_spec, _b3_spec,
              _w_spec, _v_spec, _v_spec, _v_spec,
              _w_spec, _v_spec, _v_spec, _v_spec,
              _g_spec, _g_spec, _wl_spec, _wl1_spec, _bl_spec],
    out_specs=pl.BlockSpec((_G, _OUT), lambda i: (0, 0)),
    out_shape=jax.ShapeDtypeStruct((_G, _OUT), jnp.float32),
    scratch_shapes=[pltpu.VMEM((_G, _H), jnp.float32)],
)


def kernel(x, edge_index, batch,
           W1a, b1a, g1a, t1a, W2a, b2a, g2a, t2a,
           W1b, b1b, g1b, t1b, W2b, b2b, g2b, t2b,
           Wl, bl):
    src = edge_index[0]
    dst = edge_index[1]
    b3 = batch.reshape(_GRID, 1, _B)
    r = lambda v: v.reshape(1, -1)

    agg1 = _sc_agg(x, src, dst)
    h1, m1s, cnt = _layer1(x, agg1, agg1, b3,
                           W1a, r(b1a), r(g1a), r(t1a),
                           W2a, r(b2a), r(g2a), r(t2a))
    agg2 = _sc_agg(h1, src, dst)
    wlr = Wl.reshape(2, _H, _OUT)
    return _layer2(h1, agg2, agg2, b3,
                   W1b, r(b1b), r(g1b), r(t1b),
                   W2b, r(b2b), r(g2b), r(t2b),
                   m1s, cnt, wlr, wlr, r(bl))
